# baseline (device time: 1569324 ns/iter reference)
import jax
import jax.numpy as jnp
from jax import lax
from jax.experimental import pallas as pl
from jax.experimental.pallas import tpu as pltpu

N_DEV = 16

_PERM = [0, 4, 8, 12, 13, 9, 5, 1, 2, 6, 10, 14, 15, 11, 7, 3]
_INV = [0] * N_DEV
for _r, _l in enumerate(_PERM):
    _INV[_l] = _r


def kernel(x):
    m_per, n = x.shape
    half = m_per // 2

    perm_t = jnp.array(_PERM, dtype=jnp.int32)
    inv_t = jnp.array(_INV, dtype=jnp.int32)

    my = lax.axis_index("i")
    r = inv_t[my]
    origins_fw = perm_t[(r - jnp.arange(N_DEV)) % N_DEV]
    origins_bw = perm_t[(r + jnp.arange(N_DEV)) % N_DEV]
    right = perm_t[(r + 1) % N_DEV]
    left = perm_t[(r - 1) % N_DEV]
    meta = jnp.concatenate(
        [origins_fw, origins_bw, right[None], left[None]]
    ).astype(jnp.int32)

    def body(
        meta_ref,
        x_ref,
        out_ref,
        send_f,
        recv_f,
        send_b,
        recv_b,
        copy_sem,
    ):
        right = meta_ref[2 * N_DEV]
        left = meta_ref[2 * N_DEV + 1]

        barrier_sem = pltpu.get_barrier_semaphore()
        for nbr in (left, right):
            pl.semaphore_signal(
                barrier_sem,
                inc=1,
                device_id=(nbr,),
                device_id_type=pl.DeviceIdType.MESH,
            )
        pl.semaphore_wait(barrier_sem, 2)

        own = meta_ref[0]
        cp = pltpu.make_async_copy(
            x_ref, out_ref.at[pl.ds(own * m_per, m_per)], copy_sem
        )
        cp.start()

        rdmas = []
        for h in range(N_DEV - 1):
            of = meta_ref[h]
            ob = meta_ref[N_DEV + h]
            src_f = (
                x_ref.at[pl.ds(0, half)]
                if h == 0
                else out_ref.at[pl.ds(of * m_per, half)]
            )
            src_b = (
                x_ref.at[pl.ds(half, half)]
                if h == 0
                else out_ref.at[pl.ds(ob * m_per + half, half)]
            )
            rdma_f = pltpu.make_async_remote_copy(
                src_ref=src_f,
                dst_ref=out_ref.at[pl.ds(of * m_per, half)],
                send_sem=send_f.at[h],
                recv_sem=recv_f.at[h],
                device_id=(right,),
                device_id_type=pl.DeviceIdType.MESH,
            )
            rdma_b = pltpu.make_async_remote_copy(
                src_ref=src_b,
                dst_ref=out_ref.at[pl.ds(ob * m_per + half, half)],
                send_sem=send_b.at[h],
                recv_sem=recv_b.at[h],
                device_id=(left,),
                device_id_type=pl.DeviceIdType.MESH,
            )
            rdma_f.start()
            rdma_b.start()
            rdma_f.wait_recv()
            rdma_b.wait_recv()
            rdmas.append((rdma_f, rdma_b))

        for rdma_f, rdma_b in rdmas:
            rdma_f.wait_send()
            rdma_b.wait_send()
        cp.wait()

    return pl.pallas_call(
        body,
        out_shape=jax.ShapeDtypeStruct((N_DEV * m_per, n), x.dtype),
        in_specs=[
            pl.BlockSpec(memory_space=pltpu.MemorySpace.SMEM),
            pl.BlockSpec(memory_space=pl.ANY),
        ],
        out_specs=pl.BlockSpec(memory_space=pl.ANY),
        scratch_shapes=[
            pltpu.SemaphoreType.DMA((N_DEV - 1,)),
            pltpu.SemaphoreType.DMA((N_DEV - 1,)),
            pltpu.SemaphoreType.DMA((N_DEV - 1,)),
            pltpu.SemaphoreType.DMA((N_DEV - 1,)),
            pltpu.SemaphoreType.DMA,
        ],
        compiler_params=pltpu.CompilerParams(collective_id=0),
    )(meta, x)


# device time: 1539498 ns/iter; 1.0194x vs baseline; 1.0194x over previous
import jax
import jax.numpy as jnp
from jax import lax
from jax.experimental import pallas as pl
from jax.experimental.pallas import tpu as pltpu

N_DEV = 16
SUB = 2

_PERM = [0, 4, 8, 12, 13, 9, 5, 1, 2, 6, 10, 14, 15, 11, 7, 3]
_INV = [0] * N_DEV
for _r, _l in enumerate(_PERM):
    _INV[_l] = _r


def kernel(x):
    m_per, n = x.shape
    half = m_per // 2

    perm_t = jnp.array(_PERM, dtype=jnp.int32)
    inv_t = jnp.array(_INV, dtype=jnp.int32)

    my = lax.axis_index("i")
    r = inv_t[my]
    origins_fw = perm_t[(r - jnp.arange(N_DEV)) % N_DEV]
    origins_bw = perm_t[(r + jnp.arange(N_DEV)) % N_DEV]
    right = perm_t[(r + 1) % N_DEV]
    left = perm_t[(r - 1) % N_DEV]
    meta = jnp.concatenate(
        [origins_fw, origins_bw, right[None], left[None]]
    ).astype(jnp.int32)

    def body(
        meta_ref,
        x_ref,
        out_ref,
        send_f,
        recv_f,
        send_b,
        recv_b,
        copy_sem,
    ):
        right = meta_ref[2 * N_DEV]
        left = meta_ref[2 * N_DEV + 1]

        barrier_sem = pltpu.get_barrier_semaphore()
        for nbr in (left, right):
            pl.semaphore_signal(
                barrier_sem,
                inc=1,
                device_id=(nbr,),
                device_id_type=pl.DeviceIdType.MESH,
            )
        pl.semaphore_wait(barrier_sem, 2)

        own = meta_ref[0]
        cp = pltpu.make_async_copy(
            x_ref, out_ref.at[pl.ds(own * m_per, m_per)], copy_sem
        )
        cp.start()

        quarter = half // SUB

        def make(h, s, forward):
            org = meta_ref[h] if forward else meta_ref[N_DEV + h]
            off = org * m_per + (0 if forward else half) + s * quarter
            x_off = (0 if forward else half) + s * quarter
            src = (
                x_ref.at[pl.ds(x_off, quarter)]
                if h == 0
                else out_ref.at[pl.ds(off, quarter)]
            )
            return pltpu.make_async_remote_copy(
                src_ref=src,
                dst_ref=out_ref.at[pl.ds(off, quarter)],
                send_sem=(send_f if forward else send_b).at[h, s],
                recv_sem=(recv_f if forward else recv_b).at[h, s],
                device_id=(right if forward else left,),
                device_id_type=pl.DeviceIdType.MESH,
            )

        rdmas = [
            [make(h, s, fwd) for s in range(SUB) for fwd in (True, False)]
            for h in range(N_DEV - 1)
        ]
        for r in rdmas[0]:
            r.start()
        for h in range(1, N_DEV - 1):
            for prev, nxt in zip(rdmas[h - 1], rdmas[h]):
                prev.wait_recv()
                nxt.start()
        for r in rdmas[N_DEV - 2]:
            r.wait_recv()

        for hop in rdmas:
            for r in hop:
                r.wait_send()
        cp.wait()

    return pl.pallas_call(
        body,
        out_shape=jax.ShapeDtypeStruct((N_DEV * m_per, n), x.dtype),
        in_specs=[
            pl.BlockSpec(memory_space=pltpu.MemorySpace.SMEM),
            pl.BlockSpec(memory_space=pl.ANY),
        ],
        out_specs=pl.BlockSpec(memory_space=pl.ANY),
        scratch_shapes=[
            pltpu.SemaphoreType.DMA((N_DEV - 1, SUB)),
            pltpu.SemaphoreType.DMA((N_DEV - 1, SUB)),
            pltpu.SemaphoreType.DMA((N_DEV - 1, SUB)),
            pltpu.SemaphoreType.DMA((N_DEV - 1, SUB)),
            pltpu.SemaphoreType.DMA,
        ],
        compiler_params=pltpu.CompilerParams(collective_id=0),
    )(meta, x)
